# baseline (speedup 1.0000x reference)
import functools

import jax
import jax.numpy as jnp
from jax import lax
from jax.experimental import pallas as pl
from jax.experimental.pallas import tpu as pltpu
from jax.experimental.pallas import tpu_sc as plsc


_NCHUNK = 4


def _gather_body(idx_hbm, table_hbm, out_hbm, idx_v, rows_v, gsems, wsems, *,
                 b_per_w, num_cores):
    wid = lax.axis_index("s") * num_cores + lax.axis_index("c")
    base = wid * b_per_w
    chunk = b_per_w // _NCHUNK
    pltpu.sync_copy(idx_hbm.at[pl.ds(base, b_per_w)], idx_v)
    gathers = []
    for i in range(_NCHUNK):
        gathers.append(pltpu.async_copy(
            table_hbm.at[idx_v.at[pl.ds(i * chunk, chunk)]],
            rows_v.at[pl.ds(i * chunk, chunk)], gsems[i]))
    writes = []
    for i in range(_NCHUNK):
        gathers[i].wait()
        writes.append(pltpu.async_copy(
            rows_v.at[pl.ds(i * chunk, chunk)],
            out_hbm.at[pl.ds(base + i * chunk, chunk)], wsems[i]))
    for w in writes:
        w.wait()


def kernel(inputs, weights):
    B, = inputs.shape
    V, D = weights.shape
    info = plsc.get_sparse_core_info()
    num_cores = 1
    nw = num_cores * info.num_subcores
    b_per_w = B // nw

    body = functools.partial(_gather_body, b_per_w=b_per_w,
                             num_cores=num_cores)
    run = pl.kernel(
        body,
        mesh=plsc.VectorSubcoreMesh(core_axis_name="c", subcore_axis_name="s",
                                    num_cores=num_cores),
        out_type=jax.ShapeDtypeStruct((B, D), jnp.float32),
        scratch_types=[
            pltpu.VMEM((b_per_w,), jnp.int32),
            pltpu.VMEM((b_per_w, D), jnp.float32),
            [pltpu.SemaphoreType.DMA] * _NCHUNK,
            [pltpu.SemaphoreType.DMA] * _NCHUNK,
        ],
    )
    return run(inputs, weights)


# speedup vs baseline: 1.0083x; 1.0083x over previous
import functools

import jax
import jax.numpy as jnp
from jax import lax
from jax.experimental import pallas as pl
from jax.experimental.pallas import tpu as pltpu
from jax.experimental.pallas import tpu_sc as plsc

_NCHUNK = 2


def _gather_body(idx_hbm, table_hbm, out_hbm, idx_v, rows_v, isems, gsems,
                 wsems, *, b_per_w, num_cores):
    wid = lax.axis_index("s") * num_cores + lax.axis_index("c")
    base = wid * b_per_w
    chunk = b_per_w // _NCHUNK
    idx_copies = [
        pltpu.async_copy(idx_hbm.at[pl.ds(base + i * chunk, chunk)],
                         idx_v.at[pl.ds(i * chunk, chunk)], isems[i])
        for i in range(_NCHUNK)
    ]
    gathers = []
    for i in range(_NCHUNK):
        idx_copies[i].wait()
        gathers.append(pltpu.async_copy(
            table_hbm.at[idx_v.at[pl.ds(i * chunk, chunk)]],
            rows_v.at[pl.ds(i * chunk, chunk)], gsems[i]))
    writes = []
    for i in range(_NCHUNK):
        gathers[i].wait()
        writes.append(pltpu.async_copy(
            rows_v.at[pl.ds(i * chunk, chunk)],
            out_hbm.at[pl.ds(base + i * chunk, chunk)], wsems[i]))
    for w in writes:
        w.wait()


def kernel(inputs, weights):
    B, = inputs.shape
    V, D = weights.shape
    info = plsc.get_sparse_core_info()
    nw = info.num_cores * info.num_subcores
    b_per_w = B // nw

    body = functools.partial(_gather_body, b_per_w=b_per_w,
                             num_cores=info.num_cores)
    run = pl.kernel(
        body,
        mesh=plsc.VectorSubcoreMesh(core_axis_name="c", subcore_axis_name="s"),
        out_type=jax.ShapeDtypeStruct((B, D), jnp.float32),
        scratch_types=[
            pltpu.VMEM((b_per_w,), jnp.int32),
            pltpu.VMEM((b_per_w, D), jnp.float32),
            [pltpu.SemaphoreType.DMA] * _NCHUNK,
            [pltpu.SemaphoreType.DMA] * _NCHUNK,
            [pltpu.SemaphoreType.DMA] * _NCHUNK,
        ],
    )
    return run(inputs, weights)
